# baseline (speedup 1.0000x reference)
import functools

import jax
import jax.numpy as jnp
from jax import lax
from jax.experimental import pallas as pl
from jax.experimental.pallas import tpu as pltpu
from jax.experimental.pallas import tpu_sc as plsc

N = 10000
NP = 10240
E = 320000
EC = 81
E_PAD = 32 * EC * 128
G = 64
GP = 128
NC, NS, L = 2, 16, 16
RPT = NP // (NC * NS)
RPS = NP // NS
BW = 64
BC = RPT // BW
BLK = 2048

_MESH = plsc.VectorSubcoreMesh(
    core_axis_name="c", subcore_axis_name="s", num_cores=NC, num_subcores=NS)
_SC_PARAMS = pltpu.CompilerParams(use_tc_tiling_on_sc=False)


def _sc_degree(ei_r, batch_r):
    @functools.partial(
        pl.kernel,
        out_type=(jax.ShapeDtypeStruct((NC, NP), jnp.float32),
                  jax.ShapeDtypeStruct((NC, GP), jnp.float32)),
        mesh=_MESH,
        compiler_params=_SC_PARAMS,
        scratch_types=[
            pltpu.VMEM((EC, 128), jnp.int32),
            pltpu.VMEM((BC, BW), jnp.int32),
            pltpu.VMEM((128,), jnp.float32),
            pltpu.VMEM((128,), jnp.float32),
            pltpu.VMEM_SHARED((NP,), jnp.float32),
            pltpu.VMEM_SHARED((GP,), jnp.float32),
            pltpu.SemaphoreType.DMA,
        ],
    )
    def k(ei_hbm, batch_hbm, deg_out, cnt_out, didx, bidx, ones, zbuf, dacc,
          cacc, sem):
        c = lax.axis_index("c")
        s = lax.axis_index("s")
        pltpu.sync_copy(ei_hbm.at[1, c, s], didx)
        pltpu.sync_copy(batch_hbm.at[c, s], bidx)

        @pl.loop(0, 128, step=L)
        def _(i):
            ones[pl.ds(i, L)] = jnp.ones((L,), jnp.float32)
            zbuf[pl.ds(i, L)] = jnp.zeros((L,), jnp.float32)

        @pl.loop(0, RPS, step=128)
        def _(r):
            pltpu.sync_copy(zbuf, dacc.at[pl.ds(s * RPS + r, 128)])

        @pl.when(s == 0)
        def _():
            pltpu.sync_copy(zbuf, cacc)

        plsc.subcore_barrier()

        @pl.loop(0, EC)
        def _(j):
            pltpu.async_copy(ones, dacc.at[didx.at[j]], sem, add=True)

        @pl.loop(0, BC)
        def _(j):
            pltpu.async_copy(ones.at[pl.ds(0, BW)], cacc.at[bidx.at[j]], sem,
                             add=True)

        pltpu.make_async_copy(ei_hbm.at[1, c, s], didx, sem).wait()
        pltpu.make_async_copy(batch_hbm.at[c, s], bidx, sem).wait()

        plsc.subcore_barrier()
        pltpu.sync_copy(dacc.at[pl.ds(s * RPS, RPS)],
                        deg_out.at[c, pl.ds(s * RPS, RPS)])

        @pl.when(s == 0)
        def _():
            pltpu.sync_copy(cacc, cnt_out.at[c])

    return k(ei_r, batch_r)


def _sc_aggregate(y, ei_r, d, dt=jnp.bfloat16):
    zw = 32 if dt == jnp.bfloat16 else L

    @functools.partial(
        pl.kernel,
        out_type=jax.ShapeDtypeStruct((NC, NP, d), dt),
        mesh=_MESH,
        compiler_params=_SC_PARAMS,
        scratch_types=[
            pltpu.VMEM((EC, 128), jnp.int32),
            pltpu.VMEM((EC, 128), jnp.int32),
            pltpu.VMEM((2, 128, d), dt),
            pltpu.VMEM_SHARED((NP, d), dt),
            pltpu.VMEM_SHARED((NP, d), dt),
            [pltpu.SemaphoreType.DMA] * 2,
        ],
    )
    def k(y_hbm, ei_hbm, out_hbm, sidx, didx, rows,
          ysh, acc, gsem):
        c = lax.axis_index("c")
        s = lax.axis_index("s")
        pltpu.sync_copy(ei_hbm.at[0, c, s], sidx)
        pltpu.sync_copy(ei_hbm.at[1, c, s], didx)
        pltpu.sync_copy(y_hbm.at[pl.ds(s * RPS, RPS)],
                        ysh.at[pl.ds(s * RPS, RPS)])

        @pl.loop(0, 128)
        def _(r):
            for colk in range(d // zw):
                rows[0, r, pl.ds(colk * zw, zw)] = jnp.zeros((zw,), dt)

        @pl.loop(0, RPS, step=128)
        def _(r):
            pltpu.sync_copy(rows.at[0], acc.at[pl.ds(s * RPS + r, 128)])

        plsc.subcore_barrier()

        pltpu.make_async_copy(ysh.at[sidx.at[0]], rows.at[0], gsem[0]).start()

        @pl.loop(0, (EC - 1) // 2)
        def _(p):
            j0 = p * 2
            j1 = j0 + 1
            pltpu.make_async_copy(ysh.at[sidx.at[j0]], rows.at[0],
                                  gsem[0]).wait()
            pltpu.make_async_copy(ysh.at[sidx.at[j1]], rows.at[1],
                                  gsem[1]).start()
            pltpu.sync_copy(rows.at[0], acc.at[didx.at[j0]], add=True)
            pltpu.make_async_copy(ysh.at[sidx.at[j1]], rows.at[1],
                                  gsem[1]).wait()
            pltpu.make_async_copy(ysh.at[sidx.at[j0 + 2]], rows.at[0],
                                  gsem[0]).start()
            pltpu.sync_copy(rows.at[1], acc.at[didx.at[j1]], add=True)

        pltpu.make_async_copy(ysh.at[sidx.at[EC - 1]], rows.at[0],
                              gsem[0]).wait()
        pltpu.sync_copy(rows.at[0], acc.at[didx.at[EC - 1]], add=True)

        plsc.subcore_barrier()
        pltpu.sync_copy(acc.at[pl.ds(s * RPS, RPS)],
                        out_hbm.at[c, pl.ds(s * RPS, RPS)])

    return k(y, ei_r)


def _tc_matmul(x, w):
    dout = w.shape[1]

    def body(x_ref, w_ref, o_ref):
        o_ref[...] = jnp.dot(x_ref[...], w_ref[...],
                             preferred_element_type=jnp.float32)

    return pl.pallas_call(
        body,
        grid=(NP // BLK,),
        in_specs=[pl.BlockSpec((BLK, x.shape[1]), lambda i: (i, 0)),
                  pl.BlockSpec(w.shape, lambda i: (0, 0))],
        out_specs=pl.BlockSpec((BLK, dout), lambda i: (i, 0)),
        out_shape=jax.ShapeDtypeStruct((NP, dout), jnp.float32),
    )(x, w)


def _tc_scale(deg0, deg1, t):
    d = t.shape[1]

    def body(d0, d1, t_ref, o_ref):
        dinv = lax.rsqrt(d0[...] + d1[...] + 1.0)
        o_ref[...] = (t_ref[...] * dinv).astype(jnp.bfloat16)

    return pl.pallas_call(
        body,
        grid=(NP // BLK,),
        in_specs=[pl.BlockSpec((BLK, 1), lambda i: (i, 0)),
                  pl.BlockSpec((BLK, 1), lambda i: (i, 0)),
                  pl.BlockSpec((BLK, d), lambda i: (i, 0))],
        out_specs=pl.BlockSpec((BLK, d), lambda i: (i, 0)),
        out_shape=jax.ShapeDtypeStruct((NP, d), jnp.bfloat16),
    )(deg0, deg1, t)


def _tc_layer1(a_p, t1, deg0, deg1, b1):
    d = t1.shape[1]

    def body(a_ref, t_ref, d0, d1, b_ref, z_ref, u_ref):
        dinv = lax.rsqrt(d0[...] + d1[...] + 1.0)
        a = a_ref[0].astype(jnp.float32) + a_ref[1].astype(jnp.float32)
        z = jnp.maximum(a * dinv + t_ref[...] * (dinv * dinv) + b_ref[...], 0.0)
        z_ref[...] = z
        u_ref[...] = (z * dinv).astype(jnp.bfloat16)

    return pl.pallas_call(
        body,
        grid=(NP // BLK,),
        in_specs=[pl.BlockSpec((NC, BLK, d), lambda i: (0, i, 0)),
                  pl.BlockSpec((BLK, d), lambda i: (i, 0)),
                  pl.BlockSpec((BLK, 1), lambda i: (i, 0)),
                  pl.BlockSpec((BLK, 1), lambda i: (i, 0)),
                  pl.BlockSpec((1, d), lambda i: (0, 0))],
        out_specs=[pl.BlockSpec((BLK, d), lambda i: (i, 0)),
                   pl.BlockSpec((BLK, d), lambda i: (i, 0))],
        out_shape=[jax.ShapeDtypeStruct((NP, d), jnp.float32),
                   jax.ShapeDtypeStruct((NP, d), jnp.bfloat16)],
    )(a_p, t1, deg0, deg1, b1)


def _tc_layer23(a_p, z1, deg0, deg1, w2, b2, w3):
    def body(a_ref, z1_ref, d0, d1, w2_ref, b2_ref, w3_ref,
             y3_ref, t3_ref):
        dinv = lax.rsqrt(d0[...] + d1[...] + 1.0)
        a = a_ref[0].astype(jnp.float32) + a_ref[1].astype(jnp.float32)
        q = a * dinv + z1_ref[...] * (dinv * dinv)
        z2 = jnp.maximum(
            jnp.dot(q, w2_ref[...], preferred_element_type=jnp.float32)
            + b2_ref[...], 0.0)
        t3 = jnp.dot(z2, w3_ref[...], preferred_element_type=jnp.float32)
        t3_ref[...] = t3
        y3_ref[...] = (t3 * dinv).astype(jnp.bfloat16)

    return pl.pallas_call(
        body,
        grid=(NP // BLK,),
        in_specs=[pl.BlockSpec((NC, BLK, 64), lambda i: (0, i, 0)),
                  pl.BlockSpec((BLK, 64), lambda i: (i, 0)),
                  pl.BlockSpec((BLK, 1), lambda i: (i, 0)),
                  pl.BlockSpec((BLK, 1), lambda i: (i, 0)),
                  pl.BlockSpec((64, 128), lambda i: (0, 0)),
                  pl.BlockSpec((1, 128), lambda i: (0, 0)),
                  pl.BlockSpec((128, 128), lambda i: (0, 0))],
        out_specs=[pl.BlockSpec((BLK, 128), lambda i: (i, 0)),
                   pl.BlockSpec((BLK, 128), lambda i: (i, 0))],
        out_shape=[jax.ShapeDtypeStruct((NP, 128), jnp.bfloat16),
                   jax.ShapeDtypeStruct((NP, 128), jnp.float32)],
    )(a_p, z1, deg0, deg1, w2, b2, w3)


def _tc_layer3_pool_fc(a_p, t3, deg0, deg1, b3, batch_p, cnt_p, fcw, fcb):
    nblk = NP // BLK

    def body(a_ref, t_ref, d0, d1, b_ref, bat_ref, c_ref, w_ref, fb_ref,
             o_ref, s_ref):
        i = pl.program_id(0)
        dinv = lax.rsqrt(d0[...] + d1[...] + 1.0)
        a = a_ref[0].astype(jnp.float32) + a_ref[1].astype(jnp.float32)
        z = jnp.maximum(
            a * dinv + t_ref[...] * (dinv * dinv) + b_ref[...], 0.0)
        onehot = (bat_ref[...] == jax.lax.broadcasted_iota(
            jnp.int32, (BLK, G), 1)).astype(jnp.float32)
        part = jax.lax.dot_general(
            onehot, z, (((0,), (0,)), ((), ())),
            preferred_element_type=jnp.float32)

        @pl.when(i == 0)
        def _():
            s_ref[...] = part

        @pl.when(i > 0)
        def _():
            s_ref[...] += part

        @pl.when(i == nblk - 1)
        def _():
            cnt = jnp.maximum(c_ref[0] + c_ref[1], 1.0)[:G, None]
            pooled = s_ref[...] / cnt
            o_ref[...] = jnp.dot(pooled, w_ref[...],
                                 preferred_element_type=jnp.float32) + fb_ref[...]

    out, _ = pl.pallas_call(
        body,
        grid=(nblk,),
        in_specs=[pl.BlockSpec((NC, BLK, 128), lambda i: (0, i, 0)),
                  pl.BlockSpec((BLK, 128), lambda i: (i, 0)),
                  pl.BlockSpec((BLK, 1), lambda i: (i, 0)),
                  pl.BlockSpec((BLK, 1), lambda i: (i, 0)),
                  pl.BlockSpec((1, 128), lambda i: (0, 0)),
                  pl.BlockSpec((BLK, 1), lambda i: (i, 0)),
                  pl.BlockSpec((NC, GP), lambda i: (0, 0)),
                  pl.BlockSpec((128, 16), lambda i: (0, 0)),
                  pl.BlockSpec((1, 16), lambda i: (0, 0))],
        out_specs=[pl.BlockSpec((G, 16), lambda i: (0, 0)),
                   pl.BlockSpec((G, 128), lambda i: (0, 0))],
        out_shape=[jax.ShapeDtypeStruct((G, 16), jnp.float32),
                   jax.ShapeDtypeStruct((G, 128), jnp.float32)],
    )(a_p, t3, deg0, deg1, b3, batch_p, cnt_p, fcw, fcb)
    return out


def kernel(x, edge_index, batch, W1, b1, W2, b2, W3, b3, fcW, fcb):
    f32 = jnp.float32
    ei_pad = jnp.stack([jnp.zeros((E_PAD - E,), jnp.int32),
                        jnp.full((E_PAD - E,), N, jnp.int32)])
    ei_r = jnp.concatenate([edge_index, ei_pad],
                           axis=1).reshape(2, NC, NS, EC, 128)
    batch_p = jnp.concatenate([batch, jnp.full((NP - N,), G, jnp.int32)])
    batch_r = batch_p.reshape(NC, NS, BC, BW)
    x_p = jnp.concatenate([x, jnp.zeros((NP - N, x.shape[1]), f32)])

    deg_p, cnt_p = _sc_degree(ei_r, batch_r)
    t1 = _tc_matmul(x_p, W1)

    deg0 = deg_p[0][:, None]
    deg1 = deg_p[1][:, None]

    y1 = _tc_scale(deg0, deg1, t1)
    a1 = _sc_aggregate(y1, ei_r, 64)
    z1, u1 = _tc_layer1(a1, t1, deg0, deg1, b1.reshape(1, 64))
    a2 = _sc_aggregate(u1, ei_r, 64)
    y3, t3 = _tc_layer23(a2, z1, deg0, deg1, W2, b2.reshape(1, 128), W3)
    a3 = _sc_aggregate(y3, ei_r, 128)

    batch_col = batch_p[:, None]
    fcw_p = jnp.pad(fcW, ((0, 0), (0, 6)))
    fcb_p = jnp.pad(fcb, (0, 6)).reshape(1, 16)
    res = _tc_layer3_pool_fc(a3, t3, deg0, deg1, b3.reshape(1, 128),
                             batch_col, cnt_p, fcw_p, fcb_p)
    return res[:, :10]


# speedup vs baseline: 1.0092x; 1.0092x over previous
import functools

import jax
import jax.numpy as jnp
from jax import lax
from jax.experimental import pallas as pl
from jax.experimental.pallas import tpu as pltpu
from jax.experimental.pallas import tpu_sc as plsc

N = 10000
NP = 10240
E = 320000
EC = 81
E_PAD = 32 * EC * 128
G = 64
GP = 128
NC, NS, L = 2, 16, 16
RPT = NP // (NC * NS)
RPS = NP // NS
BW = 64
BC = RPT // BW
BLK = 2048

_MESH = plsc.VectorSubcoreMesh(
    core_axis_name="c", subcore_axis_name="s", num_cores=NC, num_subcores=NS)
_SC_PARAMS = pltpu.CompilerParams(use_tc_tiling_on_sc=False)


def _sc_degree(ei_r, batch_r):
    @functools.partial(
        pl.kernel,
        out_type=(jax.ShapeDtypeStruct((NC, NP), jnp.float32),
                  jax.ShapeDtypeStruct((NC, GP), jnp.float32)),
        mesh=_MESH,
        compiler_params=_SC_PARAMS,
        scratch_types=[
            pltpu.VMEM((EC, 128), jnp.int32),
            pltpu.VMEM((BC, BW), jnp.int32),
            pltpu.VMEM((128,), jnp.float32),
            pltpu.VMEM((128,), jnp.float32),
            pltpu.VMEM_SHARED((NP,), jnp.float32),
            pltpu.VMEM_SHARED((GP,), jnp.float32),
            pltpu.SemaphoreType.DMA,
        ],
    )
    def k(ei_hbm, batch_hbm, deg_out, cnt_out, didx, bidx, ones, zbuf, dacc,
          cacc, sem):
        c = lax.axis_index("c")
        s = lax.axis_index("s")
        pltpu.sync_copy(ei_hbm.at[1, c, s], didx)
        pltpu.sync_copy(batch_hbm.at[c, s], bidx)

        @pl.loop(0, 128, step=L)
        def _(i):
            ones[pl.ds(i, L)] = jnp.ones((L,), jnp.float32)
            zbuf[pl.ds(i, L)] = jnp.zeros((L,), jnp.float32)

        @pl.loop(0, RPS, step=128)
        def _(r):
            pltpu.sync_copy(zbuf, dacc.at[pl.ds(s * RPS + r, 128)])

        @pl.when(s == 0)
        def _():
            pltpu.sync_copy(zbuf, cacc)

        plsc.subcore_barrier()

        @pl.loop(0, EC)
        def _(j):
            pltpu.async_copy(ones, dacc.at[didx.at[j]], sem, add=True)

        @pl.loop(0, BC)
        def _(j):
            pltpu.async_copy(ones.at[pl.ds(0, BW)], cacc.at[bidx.at[j]], sem,
                             add=True)

        pltpu.make_async_copy(ei_hbm.at[1, c, s], didx, sem).wait()
        pltpu.make_async_copy(batch_hbm.at[c, s], bidx, sem).wait()

        plsc.subcore_barrier()
        pltpu.sync_copy(dacc.at[pl.ds(s * RPS, RPS)],
                        deg_out.at[c, pl.ds(s * RPS, RPS)])

        @pl.when(s == 0)
        def _():
            pltpu.sync_copy(cacc, cnt_out.at[c])

    return k(ei_r, batch_r)


def _sc_aggregate(y, ei_r, d, dt=jnp.bfloat16):
    zw = 32 if dt == jnp.bfloat16 else L

    @functools.partial(
        pl.kernel,
        out_type=jax.ShapeDtypeStruct((NC, NP, d), dt),
        mesh=_MESH,
        compiler_params=_SC_PARAMS,
        scratch_types=[
            pltpu.VMEM((EC, 128), jnp.int32),
            pltpu.VMEM((EC, 128), jnp.int32),
            pltpu.VMEM((2, 128, d), dt),
            pltpu.VMEM_SHARED((NP, d), dt),
            pltpu.VMEM_SHARED((NP, d), dt),
            [pltpu.SemaphoreType.DMA] * 2,
        ],
    )
    def k(y_hbm, ei_hbm, out_hbm, sidx, didx, rows,
          ysh, acc, gsem):
        c = lax.axis_index("c")
        s = lax.axis_index("s")
        pltpu.sync_copy(ei_hbm.at[0, c, s], sidx)
        pltpu.sync_copy(ei_hbm.at[1, c, s], didx)
        pltpu.sync_copy(y_hbm.at[pl.ds(s * RPS, RPS)],
                        ysh.at[pl.ds(s * RPS, RPS)])

        @pl.loop(0, 128)
        def _(r):
            for colk in range(d // zw):
                rows[0, r, pl.ds(colk * zw, zw)] = jnp.zeros((zw,), dt)

        @pl.loop(0, RPS, step=128)
        def _(r):
            pltpu.sync_copy(rows.at[0], acc.at[pl.ds(s * RPS + r, 128)])

        plsc.subcore_barrier()

        pltpu.make_async_copy(ysh.at[sidx.at[0]], rows.at[0], gsem[0]).start()

        @pl.loop(0, (EC - 1) // 2)
        def _(p):
            j0 = p * 2
            j1 = j0 + 1
            pltpu.make_async_copy(ysh.at[sidx.at[j1]], rows.at[1],
                                  gsem[1]).start()
            pltpu.make_async_copy(ysh.at[sidx.at[j0]], rows.at[0],
                                  gsem[0]).wait()
            pltpu.sync_copy(rows.at[0], acc.at[didx.at[j0]], add=True)
            pltpu.make_async_copy(ysh.at[sidx.at[j0 + 2]], rows.at[0],
                                  gsem[0]).start()
            pltpu.make_async_copy(ysh.at[sidx.at[j1]], rows.at[1],
                                  gsem[1]).wait()
            pltpu.sync_copy(rows.at[1], acc.at[didx.at[j1]], add=True)

        pltpu.make_async_copy(ysh.at[sidx.at[EC - 1]], rows.at[0],
                              gsem[0]).wait()
        pltpu.sync_copy(rows.at[0], acc.at[didx.at[EC - 1]], add=True)

        plsc.subcore_barrier()
        pltpu.sync_copy(acc.at[pl.ds(s * RPS, RPS)],
                        out_hbm.at[c, pl.ds(s * RPS, RPS)])

    return k(y, ei_r)


def _tc_matmul(x, w):
    dout = w.shape[1]

    def body(x_ref, w_ref, o_ref):
        o_ref[...] = jnp.dot(x_ref[...], w_ref[...],
                             preferred_element_type=jnp.float32)

    return pl.pallas_call(
        body,
        grid=(NP // BLK,),
        in_specs=[pl.BlockSpec((BLK, x.shape[1]), lambda i: (i, 0)),
                  pl.BlockSpec(w.shape, lambda i: (0, 0))],
        out_specs=pl.BlockSpec((BLK, dout), lambda i: (i, 0)),
        out_shape=jax.ShapeDtypeStruct((NP, dout), jnp.float32),
    )(x, w)


def _tc_scale(deg0, deg1, t):
    d = t.shape[1]

    def body(d0, d1, t_ref, o_ref):
        dinv = lax.rsqrt(d0[...] + d1[...] + 1.0)
        o_ref[...] = (t_ref[...] * dinv).astype(jnp.bfloat16)

    return pl.pallas_call(
        body,
        grid=(NP // BLK,),
        in_specs=[pl.BlockSpec((BLK, 1), lambda i: (i, 0)),
                  pl.BlockSpec((BLK, 1), lambda i: (i, 0)),
                  pl.BlockSpec((BLK, d), lambda i: (i, 0))],
        out_specs=pl.BlockSpec((BLK, d), lambda i: (i, 0)),
        out_shape=jax.ShapeDtypeStruct((NP, d), jnp.bfloat16),
    )(deg0, deg1, t)


def _tc_layer1(a_p, t1, deg0, deg1, b1):
    d = t1.shape[1]

    def body(a_ref, t_ref, d0, d1, b_ref, z_ref, u_ref):
        dinv = lax.rsqrt(d0[...] + d1[...] + 1.0)
        a = a_ref[0].astype(jnp.float32) + a_ref[1].astype(jnp.float32)
        z = jnp.maximum(a * dinv + t_ref[...] * (dinv * dinv) + b_ref[...], 0.0)
        z_ref[...] = z
        u_ref[...] = (z * dinv).astype(jnp.bfloat16)

    return pl.pallas_call(
        body,
        grid=(NP // BLK,),
        in_specs=[pl.BlockSpec((NC, BLK, d), lambda i: (0, i, 0)),
                  pl.BlockSpec((BLK, d), lambda i: (i, 0)),
                  pl.BlockSpec((BLK, 1), lambda i: (i, 0)),
                  pl.BlockSpec((BLK, 1), lambda i: (i, 0)),
                  pl.BlockSpec((1, d), lambda i: (0, 0))],
        out_specs=[pl.BlockSpec((BLK, d), lambda i: (i, 0)),
                   pl.BlockSpec((BLK, d), lambda i: (i, 0))],
        out_shape=[jax.ShapeDtypeStruct((NP, d), jnp.float32),
                   jax.ShapeDtypeStruct((NP, d), jnp.bfloat16)],
    )(a_p, t1, deg0, deg1, b1)


def _tc_layer23(a_p, z1, deg0, deg1, w2, b2, w3):
    def body(a_ref, z1_ref, d0, d1, w2_ref, b2_ref, w3_ref,
             y3_ref, t3_ref):
        dinv = lax.rsqrt(d0[...] + d1[...] + 1.0)
        a = a_ref[0].astype(jnp.float32) + a_ref[1].astype(jnp.float32)
        q = a * dinv + z1_ref[...] * (dinv * dinv)
        z2 = jnp.maximum(
            jnp.dot(q, w2_ref[...], preferred_element_type=jnp.float32)
            + b2_ref[...], 0.0)
        t3 = jnp.dot(z2, w3_ref[...], preferred_element_type=jnp.float32)
        t3_ref[...] = t3
        y3_ref[...] = (t3 * dinv).astype(jnp.bfloat16)

    return pl.pallas_call(
        body,
        grid=(NP // BLK,),
        in_specs=[pl.BlockSpec((NC, BLK, 64), lambda i: (0, i, 0)),
                  pl.BlockSpec((BLK, 64), lambda i: (i, 0)),
                  pl.BlockSpec((BLK, 1), lambda i: (i, 0)),
                  pl.BlockSpec((BLK, 1), lambda i: (i, 0)),
                  pl.BlockSpec((64, 128), lambda i: (0, 0)),
                  pl.BlockSpec((1, 128), lambda i: (0, 0)),
                  pl.BlockSpec((128, 128), lambda i: (0, 0))],
        out_specs=[pl.BlockSpec((BLK, 128), lambda i: (i, 0)),
                   pl.BlockSpec((BLK, 128), lambda i: (i, 0))],
        out_shape=[jax.ShapeDtypeStruct((NP, 128), jnp.bfloat16),
                   jax.ShapeDtypeStruct((NP, 128), jnp.float32)],
    )(a_p, z1, deg0, deg1, w2, b2, w3)


def _tc_layer3_pool_fc(a_p, t3, deg0, deg1, b3, batch_p, cnt_p, fcw, fcb):
    nblk = NP // BLK

    def body(a_ref, t_ref, d0, d1, b_ref, bat_ref, c_ref, w_ref, fb_ref,
             o_ref, s_ref):
        i = pl.program_id(0)
        dinv = lax.rsqrt(d0[...] + d1[...] + 1.0)
        a = a_ref[0].astype(jnp.float32) + a_ref[1].astype(jnp.float32)
        z = jnp.maximum(
            a * dinv + t_ref[...] * (dinv * dinv) + b_ref[...], 0.0)
        onehot = (bat_ref[...] == jax.lax.broadcasted_iota(
            jnp.int32, (BLK, G), 1)).astype(jnp.float32)
        part = jax.lax.dot_general(
            onehot, z, (((0,), (0,)), ((), ())),
            preferred_element_type=jnp.float32)

        @pl.when(i == 0)
        def _():
            s_ref[...] = part

        @pl.when(i > 0)
        def _():
            s_ref[...] += part

        @pl.when(i == nblk - 1)
        def _():
            cnt = jnp.maximum(c_ref[0] + c_ref[1], 1.0)[:G, None]
            pooled = s_ref[...] / cnt
            o_ref[...] = jnp.dot(pooled, w_ref[...],
                                 preferred_element_type=jnp.float32) + fb_ref[...]

    out, _ = pl.pallas_call(
        body,
        grid=(nblk,),
        in_specs=[pl.BlockSpec((NC, BLK, 128), lambda i: (0, i, 0)),
                  pl.BlockSpec((BLK, 128), lambda i: (i, 0)),
                  pl.BlockSpec((BLK, 1), lambda i: (i, 0)),
                  pl.BlockSpec((BLK, 1), lambda i: (i, 0)),
                  pl.BlockSpec((1, 128), lambda i: (0, 0)),
                  pl.BlockSpec((BLK, 1), lambda i: (i, 0)),
                  pl.BlockSpec((NC, GP), lambda i: (0, 0)),
                  pl.BlockSpec((128, 16), lambda i: (0, 0)),
                  pl.BlockSpec((1, 16), lambda i: (0, 0))],
        out_specs=[pl.BlockSpec((G, 16), lambda i: (0, 0)),
                   pl.BlockSpec((G, 128), lambda i: (0, 0))],
        out_shape=[jax.ShapeDtypeStruct((G, 16), jnp.float32),
                   jax.ShapeDtypeStruct((G, 128), jnp.float32)],
    )(a_p, t3, deg0, deg1, b3, batch_p, cnt_p, fcw, fcb)
    return out


def kernel(x, edge_index, batch, W1, b1, W2, b2, W3, b3, fcW, fcb):
    f32 = jnp.float32
    ei_pad = jnp.stack([jnp.zeros((E_PAD - E,), jnp.int32),
                        jnp.full((E_PAD - E,), N, jnp.int32)])
    ei_r = jnp.concatenate([edge_index, ei_pad],
                           axis=1).reshape(2, NC, NS, EC, 128)
    batch_p = jnp.concatenate([batch, jnp.full((NP - N,), G, jnp.int32)])
    batch_r = batch_p.reshape(NC, NS, BC, BW)
    x_p = jnp.concatenate([x, jnp.zeros((NP - N, x.shape[1]), f32)])

    deg_p, cnt_p = _sc_degree(ei_r, batch_r)
    t1 = _tc_matmul(x_p, W1)

    deg0 = deg_p[0][:, None]
    deg1 = deg_p[1][:, None]

    y1 = _tc_scale(deg0, deg1, t1)
    a1 = _sc_aggregate(y1, ei_r, 64)
    z1, u1 = _tc_layer1(a1, t1, deg0, deg1, b1.reshape(1, 64))
    a2 = _sc_aggregate(u1, ei_r, 64)
    y3, t3 = _tc_layer23(a2, z1, deg0, deg1, W2, b2.reshape(1, 128), W3)
    a3 = _sc_aggregate(y3, ei_r, 128)

    batch_col = batch_p[:, None]
    fcw_p = jnp.pad(fcW, ((0, 0), (0, 6)))
    fcb_p = jnp.pad(fcb, (0, 6)).reshape(1, 16)
    res = _tc_layer3_pool_fc(a3, t3, deg0, deg1, b3.reshape(1, 128),
                             batch_col, cnt_p, fcw_p, fcb_p)
    return res[:, :10]
